# baseline (speedup 1.0000x reference)
import functools

import jax
import jax.numpy as jnp
from jax import lax
from jax.experimental import pallas as pl
from jax.experimental.pallas import tpu as pltpu
from jax.experimental.pallas import tpu_sc as plsc

NCHAN = 7
NTC = 2 * NCHAN
NCS = 6 * 64 * 64
NLL = 721 * 1440
NROWS = 721
RW = 1440
NCORES, NSUB = 2, 16
NW = NCORES * NSUB
CS_T = NCS // NW
ROWS_PER_TILE = 23

_mesh = lambda: plsc.VectorSubcoreMesh(
    core_axis_name="c", subcore_axis_name="s", num_cores=NCORES,
    num_subcores=NSUB)


def _wid():
    return lax.axis_index("s") * NCORES + lax.axis_index("c")


def _in_remap(x1d, colsA, valsA):
    @functools.partial(
        pl.kernel,
        out_type=jax.ShapeDtypeStruct((NTC, NW, CS_T), jnp.float32),
        mesh=_mesh(),
        compiler_params=pltpu.CompilerParams(needs_layout_passes=False),
        scratch_types=[
            pltpu.VMEM((4 * CS_T,), jnp.int32),
            pltpu.VMEM((4 * CS_T,), jnp.int32),
            pltpu.VMEM((4 * CS_T,), jnp.float32),
            pltpu.VMEM((4 * CS_T,), jnp.float32),
            pltpu.VMEM((CS_T,), jnp.float32),
            pltpu.SemaphoreType.DMA,
        ],
    )
    def k(x_hbm, cols_hbm, vals_hbm, cs_hbm, idx_v, aidx_v, val_v, g_v,
          acc_v, sem):
        w = _wid()
        pltpu.sync_copy(cols_hbm.at[pl.ds(w * 4 * CS_T, 4 * CS_T)], idx_v)
        pltpu.sync_copy(vals_hbm.at[pl.ds(w * 4 * CS_T, 4 * CS_T)], val_v)
        iota4 = lax.iota(jnp.int32, 16) * 4

        def tr_body(i, c2):
            s = pl.ds(i * 16, 16)
            col = idx_v[s]
            h = col // 1440
            idx_v[s] = (col - h * 1440) * 721 + h
            return c2
        lax.fori_loop(0, 4 * CS_T // 16, tr_body, 0)

        def tc_body(tc, carry):
            off = tc * NLL

            def add_body(i, c2):
                s = pl.ds(i * 16, 16)
                aidx_v[s] = idx_v[s] + off
                return c2
            lax.fori_loop(0, 4 * CS_T // 16, add_body, 0)
            pltpu.async_copy(x_hbm.at[aidx_v], g_v, sem).wait()

            def red_body(i, c2):
                base = iota4 + i * 64
                g = [plsc.load_gather(g_v, [base + kk])
                     * plsc.load_gather(val_v, [base + kk])
                     for kk in range(4)]
                acc_v[pl.ds(i * 16, 16)] = (g[0] + g[1]) + (g[2] + g[3])
                return c2
            lax.fori_loop(0, CS_T // 16, red_body, 0, unroll=4)
            pltpu.sync_copy(acc_v, cs_hbm.at[tc, w])
            return carry
        lax.fori_loop(0, NTC, tc_body, 0)

    return k(x1d, colsA, valsA)


def _tc_model_body(cs_ref, lon_ref, lat_ref, lsm_ref, topo_ref, w_ref,
                   tp_ref, y_ref):
    lon = lon_ref[...]
    lat = lat_ref[...]
    sin_lat = jnp.sin(lat)
    cos_lat = jnp.cos(lat)
    cos_lon = jnp.cos(lon)
    sin_lon = jnp.sin(lon)
    rows = []
    cs = cs_ref[...]
    for t in range(2):
        cz = (tp_ref[t, 0] * sin_lat
              + cos_lat * (tp_ref[t, 1] * cos_lon - tp_ref[t, 2] * sin_lon))
        tisr = jnp.maximum(cz, 0.0) - 1.0 / jnp.pi
        rows.append(cs[t * NCHAN:(t + 1) * NCHAN, :])
        rows.append(tisr)
    rows.append(lsm_ref[...])
    rows.append((topo_ref[...] - 3724.0) / 8349.0)
    in18 = jnp.concatenate(rows, axis=0)
    y_ref[...] = jnp.dot(w_ref[...], in18,
                         preferred_element_type=jnp.float32)


def _tc_model(cs, lon, lat, lsm, topo, W, tp):
    return pl.pallas_call(
        _tc_model_body,
        out_shape=jax.ShapeDtypeStruct((NTC, NCS), jnp.float32),
        in_specs=[
            pl.BlockSpec(memory_space=pltpu.VMEM),
            pl.BlockSpec(memory_space=pltpu.VMEM),
            pl.BlockSpec(memory_space=pltpu.VMEM),
            pl.BlockSpec(memory_space=pltpu.VMEM),
            pl.BlockSpec(memory_space=pltpu.VMEM),
            pl.BlockSpec(memory_space=pltpu.VMEM),
            pl.BlockSpec(memory_space=pltpu.SMEM),
        ],
        out_specs=pl.BlockSpec(memory_space=pltpu.VMEM),
    )(cs, lon, lat, lsm, topo, W, tp)


_GROUPS = ((0, 4), (4, 4), (8, 4), (12, 2))


def _out_remap(y1d, ocols, ovals):
    HR = RW // 2
    NHR = NROWS * 2
    JT = 46
    VPC = HR // 16

    @functools.partial(
        pl.kernel,
        out_type=jax.ShapeDtypeStruct((NTC * NLL,), jnp.float32),
        mesh=_mesh(),
        compiler_params=pltpu.CompilerParams(needs_layout_passes=False),
        scratch_types=[
            pltpu.VMEM((NCS,), jnp.float32),
            pltpu.VMEM((NCS,), jnp.float32),
            pltpu.VMEM((NCS,), jnp.float32),
            pltpu.VMEM((NCS,), jnp.float32),
            pltpu.VMEM((4 * HR,), jnp.int32),
            pltpu.VMEM((4 * HR,), jnp.int32),
            pltpu.VMEM((4 * HR,), jnp.float32),
            pltpu.VMEM((4 * HR,), jnp.float32),
            pltpu.VMEM((HR,), jnp.float32), pltpu.VMEM((HR,), jnp.float32),
            pltpu.VMEM((HR,), jnp.float32), pltpu.VMEM((HR,), jnp.float32),
            pltpu.VMEM((HR,), jnp.float32), pltpu.VMEM((HR,), jnp.float32),
            pltpu.VMEM((HR,), jnp.float32), pltpu.VMEM((HR,), jnp.float32),
            pltpu.SemaphoreType.DMA, pltpu.SemaphoreType.DMA,
            pltpu.SemaphoreType.DMA, pltpu.SemaphoreType.DMA,
        ],
    )
    def k(y_hbm, oc_hbm, ov_hbm, out_hbm, t0, t1, t2, t3, ia, ib, va, vb,
          oa0, oa1, oa2, oa3, ob0, ob1, ob2, ob3, sla, slb, ssa, ssb):
        w = _wid()
        tbls = (t0, t1, t2, t3)
        idxs = (ia, ib)
        vals = (va, vb)
        obufs = ((oa0, oa1, oa2, oa3), (ob0, ob1, ob2, ob3))
        sl = (sla, slb)
        ss = (ssa, ssb)
        iota4 = lax.iota(jnp.int32, 16) * 4

        def chunk_of(j):
            return jnp.minimum(j * NW + w, NHR - 1)

        def fire_loads(j, st):
            c = chunk_of(j)
            pltpu.async_copy(oc_hbm.at[pl.ds(c * 4 * HR, 4 * HR)],
                             idxs[st], sl[st])
            pltpu.async_copy(ov_hbm.at[pl.ds(c * 4 * HR, 4 * HR)],
                             vals[st], sl[st])

        def wait_loads(st):
            pltpu.make_async_copy(oc_hbm.at[pl.ds(0, 4 * HR)],
                                  idxs[st], sl[st]).wait()
            pltpu.make_async_copy(ov_hbm.at[pl.ds(0, 4 * HR)],
                                  vals[st], sl[st]).wait()

        for (c0, gn) in _GROUPS:
            for cl in range(gn):
                pltpu.sync_copy(y_hbm.at[pl.ds((c0 + cl) * NCS, NCS)],
                                tbls[cl])
            fire_loads(0, 0)

            def pair_body(jj, carry):
                for ph in range(2):
                    j = 2 * jj + ph
                    st = ph
                    fire_loads(j + 1, 1 - st)
                    wait_loads(st)

                    @pl.when(j >= 2)
                    def _():
                        for cl in range(gn):
                            pltpu.make_async_copy(
                                obufs[st][cl],
                                out_hbm.at[pl.ds(cl * HR, HR)],
                                ss[st]).wait()
                    idx_v = idxs[st]
                    val_v = vals[st]

                    def vec_body(v, c2):
                        s = pl.ds(v * 16, 16)
                        base = iota4 + v * 64
                        ivs = [plsc.load_gather(idx_v, [base + kk])
                               for kk in range(4)]
                        vvs = [plsc.load_gather(val_v, [base + kk])
                               for kk in range(4)]
                        for cl in range(gn):
                            g = [plsc.load_gather(tbls[cl], [ivs[kk]])
                                 * vvs[kk] for kk in range(4)]
                            obufs[st][cl][s] = (g[0] + g[1]) + (g[2] + g[3])
                        return c2
                    lax.fori_loop(0, VPC, vec_body, 0, unroll=5)
                    c = chunk_of(j)
                    for cl in range(gn):
                        pltpu.async_copy(
                            obufs[st][cl],
                            out_hbm.at[pl.ds((c0 + cl) * NLL + c * HR, HR)],
                            ss[st])
                return carry
            lax.fori_loop(0, JT // 2, pair_body, 0)
            wait_loads(JT & 1)
            for jt in (JT - 2, JT - 1):
                st = jt & 1
                for cl in range(gn):
                    pltpu.make_async_copy(
                        obufs[st][cl],
                        out_hbm.at[pl.ds(cl * HR, HR)], ss[st]).wait()

    return k(y1d, ocols, ovals)


def kernel(x, time_hours, in_rows, in_cols, in_vals, out_rows, out_cols,
           out_vals, longrid, latgrid, lsm, topographic_height, W_model):
    del in_rows, out_rows
    f32 = jnp.float32

    x1d = x.transpose(0, 1, 2, 4, 3).reshape(NTC * NLL).astype(f32)
    colsA = in_cols.astype(jnp.int32)
    valsA = in_vals.astype(f32)
    ocols = out_cols.astype(jnp.int32)
    ovals = out_vals.astype(f32)

    th = jnp.asarray(time_hours, f32)
    tps = []
    for t in range(2):
        tt = th + 6.0 * t
        day = (tt / 24.0) % 365.25
        hour = tt % 24.0
        decl = (23.45 * jnp.pi / 180.0) * jnp.sin(
            2.0 * jnp.pi * (day - 81.0) / 365.25)
        h0 = (hour - 12.0) * jnp.pi / 12.0
        tps.append(jnp.stack([jnp.sin(decl),
                              jnp.cos(decl) * jnp.cos(h0),
                              jnp.cos(decl) * jnp.sin(h0)]))
    tp = jnp.stack(tps)

    lon = longrid.reshape(1, NCS).astype(f32)
    lat = latgrid.reshape(1, NCS).astype(f32)
    lsm2 = lsm.reshape(1, NCS).astype(f32)
    topo2 = topographic_height.reshape(1, NCS).astype(f32)

    cs = _in_remap(x1d, colsA, valsA).reshape(NTC, NCS)

    y = _tc_model(cs, lon, lat, lsm2, topo2, W_model.astype(f32), tp)

    out1d = _out_remap(y.reshape(NTC * NCS), ocols, ovals)
    return out1d.reshape(1, 2, NCHAN, 721, 1440)


# speedup vs baseline: 1.2160x; 1.2160x over previous
import functools

import jax
import jax.numpy as jnp
from jax import lax
from jax.experimental import pallas as pl
from jax.experimental.pallas import tpu as pltpu
from jax.experimental.pallas import tpu_sc as plsc

NCHAN = 7
NTC = 2 * NCHAN
NCS = 6 * 64 * 64
NLL = 721 * 1440
NROWS = 721
RW = 1440
NCORES, NSUB = 2, 16
NW = NCORES * NSUB
CS_T = NCS // NW
ROWS_PER_TILE = 23

_mesh = lambda: plsc.VectorSubcoreMesh(
    core_axis_name="c", subcore_axis_name="s", num_cores=NCORES,
    num_subcores=NSUB)


def _wid():
    return lax.axis_index("s") * NCORES + lax.axis_index("c")


def _in_remap(x1d, colsA, valsA):
    @functools.partial(
        pl.kernel,
        out_type=jax.ShapeDtypeStruct((NTC, NW, CS_T), jnp.float32),
        mesh=_mesh(),
        compiler_params=pltpu.CompilerParams(needs_layout_passes=False),
        scratch_types=[
            pltpu.VMEM((4 * CS_T,), jnp.int32),
            pltpu.VMEM((4 * CS_T,), jnp.int32),
            pltpu.VMEM((4 * CS_T,), jnp.float32),
            pltpu.VMEM((4 * CS_T,), jnp.float32),
            pltpu.VMEM((CS_T,), jnp.float32),
            pltpu.SemaphoreType.DMA,
        ],
    )
    def k(x_hbm, cols_hbm, vals_hbm, cs_hbm, idx_v, aidx_v, val_v, g_v,
          acc_v, sem):
        w = _wid()
        pltpu.sync_copy(cols_hbm.at[pl.ds(w * 4 * CS_T, 4 * CS_T)], idx_v)
        pltpu.sync_copy(vals_hbm.at[pl.ds(w * 4 * CS_T, 4 * CS_T)], val_v)
        iota4 = lax.iota(jnp.int32, 16) * 4

        def tr_body(i, c2):
            s = pl.ds(i * 16, 16)
            col = idx_v[s]
            h = col // 1440
            idx_v[s] = (col - h * 1440) * 721 + h
            return c2
        lax.fori_loop(0, 4 * CS_T // 16, tr_body, 0)

        def tc_body(tc, carry):
            off = tc * NLL

            def add_body(i, c2):
                s = pl.ds(i * 16, 16)
                aidx_v[s] = idx_v[s] + off
                return c2
            lax.fori_loop(0, 4 * CS_T // 16, add_body, 0)
            pltpu.async_copy(x_hbm.at[aidx_v], g_v, sem).wait()

            def red_body(i, c2):
                base = iota4 + i * 64
                g = [plsc.load_gather(g_v, [base + kk])
                     * plsc.load_gather(val_v, [base + kk])
                     for kk in range(4)]
                acc_v[pl.ds(i * 16, 16)] = (g[0] + g[1]) + (g[2] + g[3])
                return c2
            lax.fori_loop(0, CS_T // 16, red_body, 0, unroll=4)
            pltpu.sync_copy(acc_v, cs_hbm.at[tc, w])
            return carry
        lax.fori_loop(0, NTC, tc_body, 0)

    return k(x1d, colsA, valsA)


def _tc_model_body(cs_ref, lon_ref, lat_ref, lsm_ref, topo_ref, w_ref,
                   tp_ref, y_ref):
    lon = lon_ref[...]
    lat = lat_ref[...]
    sin_lat = jnp.sin(lat)
    cos_lat = jnp.cos(lat)
    cos_lon = jnp.cos(lon)
    sin_lon = jnp.sin(lon)
    rows = []
    cs = cs_ref[...]
    for t in range(2):
        cz = (tp_ref[t, 0] * sin_lat
              + cos_lat * (tp_ref[t, 1] * cos_lon - tp_ref[t, 2] * sin_lon))
        tisr = jnp.maximum(cz, 0.0) - 1.0 / jnp.pi
        rows.append(cs[t * NCHAN:(t + 1) * NCHAN, :])
        rows.append(tisr)
    rows.append(lsm_ref[...])
    rows.append((topo_ref[...] - 3724.0) / 8349.0)
    in18 = jnp.concatenate(rows, axis=0)
    y = jnp.dot(w_ref[...], in18, preferred_element_type=jnp.float32)
    y3 = y.reshape(NTC // 2, 2, NCS)
    lo = jax.lax.bitcast_convert_type(
        y3[:, 0, :].astype(jnp.bfloat16), jnp.uint16).astype(jnp.uint32)
    hi = jax.lax.bitcast_convert_type(
        y3[:, 1, :].astype(jnp.bfloat16), jnp.uint16).astype(jnp.uint32)
    y_ref[...] = jax.lax.bitcast_convert_type(lo | (hi << 16), jnp.int32)


def _tc_model(cs, lon, lat, lsm, topo, W, tp):
    return pl.pallas_call(
        _tc_model_body,
        out_shape=jax.ShapeDtypeStruct((NTC // 2, NCS), jnp.int32),
        in_specs=[
            pl.BlockSpec(memory_space=pltpu.VMEM),
            pl.BlockSpec(memory_space=pltpu.VMEM),
            pl.BlockSpec(memory_space=pltpu.VMEM),
            pl.BlockSpec(memory_space=pltpu.VMEM),
            pl.BlockSpec(memory_space=pltpu.VMEM),
            pl.BlockSpec(memory_space=pltpu.VMEM),
            pl.BlockSpec(memory_space=pltpu.SMEM),
        ],
        out_specs=pl.BlockSpec(memory_space=pltpu.VMEM),
    )(cs, lon, lat, lsm, topo, W, tp)


_GROUPS = ((0, 4), (4, 3))


def _out_remap(y1d, ocols, ovals):
    HR = RW // 2
    NHR = NROWS * 2
    JT = 46
    VPC = HR // 16

    @functools.partial(
        pl.kernel,
        out_type=jax.ShapeDtypeStruct((NTC * NLL,), jnp.float32),
        mesh=_mesh(),
        compiler_params=pltpu.CompilerParams(needs_layout_passes=False),
        scratch_types=(
            [pltpu.VMEM((NCS,), jnp.int32)] * 4
            + [pltpu.VMEM((4 * HR,), jnp.int32)] * 2
            + [pltpu.VMEM((4 * HR,), jnp.float32)] * 2
            + [pltpu.VMEM((HR,), jnp.float32)] * 16
            + [pltpu.SemaphoreType.DMA] * 4
        ),
    )
    def k(y_hbm, oc_hbm, ov_hbm, out_hbm, t0, t1, t2, t3, ia, ib, va, vb,
          *rest):
        obuf_flat, (sla, slb, ssa, ssb) = rest[:16], rest[16:]
        w = _wid()
        tbls = (t0, t1, t2, t3)
        idxs = (ia, ib)
        vals = (va, vb)
        obufs = (obuf_flat[:8], obuf_flat[8:])
        sl = (sla, slb)
        ss = (ssa, ssb)
        iota4 = lax.iota(jnp.int32, 16) * 4
        himask = jnp.full((16,), -65536, jnp.int32)

        def chunk_of(j):
            return jnp.minimum(j * NW + w, NHR - 1)

        def fire_loads(j, st):
            c = chunk_of(j)
            pltpu.async_copy(oc_hbm.at[pl.ds(c * 4 * HR, 4 * HR)],
                             idxs[st], sl[st])
            pltpu.async_copy(ov_hbm.at[pl.ds(c * 4 * HR, 4 * HR)],
                             vals[st], sl[st])

        def wait_loads(st):
            pltpu.make_async_copy(oc_hbm.at[pl.ds(0, 4 * HR)],
                                  idxs[st], sl[st]).wait()
            pltpu.make_async_copy(ov_hbm.at[pl.ds(0, 4 * HR)],
                                  vals[st], sl[st]).wait()

        for (p0, gn) in _GROUPS:
            nch = 2 * gn
            for pp in range(gn):
                pltpu.sync_copy(y_hbm.at[pl.ds((p0 + pp) * NCS, NCS)],
                                tbls[pp])
            fire_loads(0, 0)

            def pair_body(jj, carry):
                for ph in range(2):
                    j = 2 * jj + ph
                    st = ph
                    fire_loads(j + 1, 1 - st)
                    wait_loads(st)

                    @pl.when(j >= 2)
                    def _():
                        for cl in range(nch):
                            pltpu.make_async_copy(
                                obufs[st][cl],
                                out_hbm.at[pl.ds(cl * HR, HR)],
                                ss[st]).wait()
                    idx_v = idxs[st]
                    val_v = vals[st]

                    def vec_body(v, c2):
                        s = pl.ds(v * 16, 16)
                        base = iota4 + v * 64
                        ivs = [plsc.load_gather(idx_v, [base + kk])
                               for kk in range(4)]
                        vvs = [plsc.load_gather(val_v, [base + kk])
                               for kk in range(4)]
                        for pp in range(gn):
                            pv = [plsc.load_gather(tbls[pp], [ivs[kk]])
                                  for kk in range(4)]
                            ga = [plsc.bitcast(pv[kk] << 16, jnp.float32)
                                  * vvs[kk] for kk in range(4)]
                            gb = [plsc.bitcast(pv[kk] & himask, jnp.float32)
                                  * vvs[kk] for kk in range(4)]
                            obufs[st][2 * pp][s] = ((ga[0] + ga[1])
                                                    + (ga[2] + ga[3]))
                            obufs[st][2 * pp + 1][s] = ((gb[0] + gb[1])
                                                        + (gb[2] + gb[3]))
                        return c2
                    lax.fori_loop(0, VPC, vec_body, 0, unroll=2)
                    c = chunk_of(j)
                    for cl in range(nch):
                        pltpu.async_copy(
                            obufs[st][cl],
                            out_hbm.at[pl.ds((2 * p0 + cl) * NLL + c * HR,
                                             HR)],
                            ss[st])
                return carry
            lax.fori_loop(0, JT // 2, pair_body, 0)
            wait_loads(JT & 1)
            for jt in (JT - 2, JT - 1):
                st = jt & 1
                for cl in range(nch):
                    pltpu.make_async_copy(
                        obufs[st][cl],
                        out_hbm.at[pl.ds(cl * HR, HR)], ss[st]).wait()

    return k(y1d, ocols, ovals)


def kernel(x, time_hours, in_rows, in_cols, in_vals, out_rows, out_cols,
           out_vals, longrid, latgrid, lsm, topographic_height, W_model):
    del in_rows, out_rows
    f32 = jnp.float32

    x1d = x.transpose(0, 1, 2, 4, 3).reshape(NTC * NLL).astype(f32)
    colsA = in_cols.astype(jnp.int32)
    valsA = in_vals.astype(f32)
    ocols = out_cols.astype(jnp.int32)
    ovals = out_vals.astype(f32)

    th = jnp.asarray(time_hours, f32)
    tps = []
    for t in range(2):
        tt = th + 6.0 * t
        day = (tt / 24.0) % 365.25
        hour = tt % 24.0
        decl = (23.45 * jnp.pi / 180.0) * jnp.sin(
            2.0 * jnp.pi * (day - 81.0) / 365.25)
        h0 = (hour - 12.0) * jnp.pi / 12.0
        tps.append(jnp.stack([jnp.sin(decl),
                              jnp.cos(decl) * jnp.cos(h0),
                              jnp.cos(decl) * jnp.sin(h0)]))
    tp = jnp.stack(tps)

    lon = longrid.reshape(1, NCS).astype(f32)
    lat = latgrid.reshape(1, NCS).astype(f32)
    lsm2 = lsm.reshape(1, NCS).astype(f32)
    topo2 = topographic_height.reshape(1, NCS).astype(f32)

    cs = _in_remap(x1d, colsA, valsA).reshape(NTC, NCS)

    y = _tc_model(cs, lon, lat, lsm2, topo2, W_model.astype(f32), tp)

    out1d = _out_remap(y.reshape((NTC // 2) * NCS), ocols, ovals)
    return out1d.reshape(1, 2, NCHAN, 721, 1440)
